# Initial kernel scaffold; baseline (speedup 1.0000x reference)
import jax
import jax.numpy as jnp
from jax.experimental import pallas as pl


def kernel(x, edge_index, edge_weight, h_0, c_0, ggc_w, gru_w_ih, gru_w_hh, gru_b_ih, gru_b_hh, lstm_w_ih, lstm_w_hh, lstm_b_ih, lstm_b_hh, lin_w, lin_b):
    raise NotImplementedError("write your pallas kernel here")


# speedup vs baseline: 9.9419x; 9.9419x over previous
import functools

import jax
import jax.numpy as jnp
from jax import lax
from jax.experimental import pallas as pl
from jax.experimental.pallas import tpu as pltpu
from jax.experimental.pallas import tpu_sc as plsc

MCW = 128


@functools.lru_cache(maxsize=None)
def _sc_agg(n, n_pad, hid, mc_total, mc_real, with_cnt, cnt_stride):
  mesh = plsc.VectorSubcoreMesh(core_axis_name="c", subcore_axis_name="s")
  mc_per_tile = mc_total // 32
  assert mc_per_tile % 2 == 0 and mc_total % 32 == 0
  rows_per_tile = n_pad // 16
  nh16 = hid // 16

  out_type = [jax.ShapeDtypeStruct((2, n_pad, hid), jnp.float32)]
  if with_cnt:
    out_type.append(
        jax.ShapeDtypeStruct((2, 1, 16 * cnt_stride), jnp.float32))

  scratch = (
      pltpu.VMEM((mc_per_tile, MCW), jnp.int32),
      pltpu.VMEM((mc_per_tile, MCW), jnp.int32),
      pltpu.VMEM((mc_per_tile, MCW), jnp.float32),
      pltpu.VMEM((mc_per_tile, MCW), jnp.float32),
      pltpu.VMEM((2, MCW, hid), jnp.float32),
      pltpu.VMEM((2, MCW, hid), jnp.float32),
      pltpu.VMEM((rows_per_tile, hid), jnp.float32),
      pltpu.VMEM((cnt_stride,), jnp.float32),
      pltpu.VMEM_SHARED((n_pad, hid), jnp.float32),
      pltpu.VMEM_SHARED((16 * cnt_stride,), jnp.float32),
      pltpu.SemaphoreType.DMA,
      pltpu.SemaphoreType.DMA,
      pltpu.SemaphoreType.DMA,
      pltpu.SemaphoreType.DMA,
  )

  def body(*refs):
    if with_cnt:
      (src2_h, dst2_h, ew2_h, val2_h, h_h, part_o, cntp_o) = refs[:7]
      sc = refs[7:]
    else:
      (src2_h, dst2_h, ew2_h, h_h, part_o) = refs[:5]
      cntp_o = val2_h = None
      sc = refs[5:]
    (src_v, dst_v, ew_v, val_v, rows_v, msg_v, zero_v, zline_v, acc_sp,
     cnt_sp, gsem0, gsem1, ssem0, ssem1) = sc
    gsems = (gsem0, gsem1)
    ssems = (ssem0, ssem1)

    c = lax.axis_index("c")
    s = lax.axis_index("s")
    wid = c * 16 + s
    mc0 = wid * mc_per_tile
    r_real = jnp.clip(mc_real - mc0, 0, mc_per_tile)

    zv = jnp.zeros((16,), jnp.float32)

    def zb(i, carry):
      for j in range(nh16):
        zero_v[i, pl.ds(16 * j, 16)] = zv
      return carry

    lax.fori_loop(0, rows_per_tile, zb, 0)
    pltpu.sync_copy(zero_v, acc_sp.at[pl.ds(s * rows_per_tile, rows_per_tile), :])
    if with_cnt:
      def zc(i, carry):
        zline_v[pl.ds(16 * i, 16)] = zv
        return carry

      lax.fori_loop(0, cnt_stride // 16, zc, 0)
      pltpu.sync_copy(zline_v, cnt_sp.at[pl.ds(s * cnt_stride, cnt_stride)])

    pltpu.sync_copy(src2_h.at[pl.ds(mc0, mc_per_tile), :], src_v)
    pltpu.sync_copy(dst2_h.at[pl.ds(mc0, mc_per_tile), :], dst_v)
    pltpu.sync_copy(ew2_h.at[pl.ds(mc0, mc_per_tile), :], ew_v)
    if with_cnt:
      pltpu.sync_copy(val2_h.at[pl.ds(mc0, mc_per_tile), :], val_v)

    plsc.subcore_barrier()

    @pl.when(0 < r_real)
    def _():
      pltpu.async_copy(h_h.at[src_v.at[0]], rows_v.at[0], gsems[0])

    def do_mc(m, slot):
      nxt = m + 1

      @pl.when(m < r_real)
      def _():
        pltpu.make_async_copy(h_h.at[src_v.at[m]], rows_v.at[slot],
                              gsems[slot]).wait()

      @pl.when(nxt < r_real)
      def _():
        pltpu.async_copy(h_h.at[src_v.at[nxt]], rows_v.at[1 - slot],
                         gsems[1 - slot])

      @pl.when((m >= 2) & (m < r_real))
      def _():
        pltpu.make_async_copy(msg_v.at[slot], acc_sp.at[dst_v.at[m - 2]],
                              ssems[slot]).wait()

      @pl.when(m < r_real)
      def _():
        for b in range(MCW // 16):
          ew_vec = ew_v[m, pl.ds(16 * b, 16)]
          for l in range(16):
            e = 16 * b + l
            w = ew_vec[l]
            for j in range(nh16):
              msg_v[slot, e, pl.ds(16 * j, 16)] = (
                  rows_v[slot, e, pl.ds(16 * j, 16)] * w)
        pltpu.async_copy(msg_v.at[slot], acc_sp.at[dst_v.at[m]], ssems[slot],
                         add=True)
        if with_cnt:
          pltpu.sync_copy(val_v.at[m], cnt_sp.at[dst_v.at[m]], add=True)

    def loop_body(i, carry):
      do_mc(2 * i, 0)
      do_mc(2 * i + 1, 1)
      return carry

    lax.fori_loop(0, mc_per_tile // 2, loop_body, 0)

    for slot in (0, 1):
      lastm = r_real - 1 - ((r_real - 1 - slot) % 2)

      @pl.when(lastm >= 0)
      def _():
        pltpu.make_async_copy(
            msg_v.at[slot], acc_sp.at[dst_v.at[jnp.maximum(lastm, 0)]],
            ssems[slot]).wait()

    plsc.subcore_barrier()

    pltpu.sync_copy(acc_sp.at[pl.ds(s * rows_per_tile, rows_per_tile), :],
                    part_o.at[c, pl.ds(s * rows_per_tile, rows_per_tile), :])
    if with_cnt:
      @pl.when(s == 0)
      def _():
        pltpu.sync_copy(cnt_sp, cntp_o.at[c, 0])

  return pl.kernel(body, out_type=tuple(out_type), mesh=mesh,
                   scratch_types=scratch,
                   compiler_params=pltpu.CompilerParams(
                       use_tc_tiling_on_sc=False))


def _gru_block(part0, part1, inv, h, w_g, wih, whh, bih, bhh):
  agg = ((part0 + part1) * inv) @ w_g
  r = jax.nn.sigmoid(agg @ wih[0] + bih[0] + h @ whh[0] + bhh[0])
  z = jax.nn.sigmoid(agg @ wih[1] + bih[1] + h @ whh[1] + bhh[1])
  nn = jnp.tanh(agg @ wih[2] + bih[2] + r * (h @ whh[2] + bhh[2]))
  return (1.0 - z) * nn + z * h


def _tc_layer1(n, hid, part, cntp, h, w_g, wih, whh, bih, bhh):
  def f(part_ref, cntp_ref, h_ref, wg_ref, wih_ref, whh_ref, bih_ref,
        bhh_ref, h_out, inv_out):
    cnt = cntp_ref[0] + cntp_ref[1]
    inv_full = 1.0 / jnp.clip(cnt, 1.0, None)
    inv_out[...] = inv_full
    inv = inv_full[:n]
    h_out[...] = _gru_block(part_ref[0][:n], part_ref[1][:n], inv, h_ref[...],
                            wg_ref[...], wih_ref, whh_ref, bih_ref, bhh_ref)

  np_ = cntp.shape[1]
  return pl.pallas_call(
      f,
      out_shape=(jax.ShapeDtypeStruct((n, hid), jnp.float32),
                 jax.ShapeDtypeStruct((np_, 1), jnp.float32)),
  )(part, cntp, h, w_g, wih, whh, bih, bhh)


def _tc_layer2(n, hid, part, inv_full, h, w_g, wih, whh, bih, bhh):
  def f(part_ref, inv_ref, h_ref, wg_ref, wih_ref, whh_ref, bih_ref,
        bhh_ref, h_out):
    inv = inv_ref[:n]
    h_out[...] = _gru_block(part_ref[0][:n], part_ref[1][:n], inv, h_ref[...],
                            wg_ref[...], wih_ref, whh_ref, bih_ref, bhh_ref)

  return pl.pallas_call(
      f, out_shape=jax.ShapeDtypeStruct((n, hid), jnp.float32),
  )(part, inv_full, h, w_g, wih, whh, bih, bhh)


def _tc_layer3(n, hid, out_dim, part, inv_full, h, w_g, wih, whh, bih, bhh,
               h0, c0, lwih, lwhh, lbih, lbhh, linw, linb):
  def f(part_ref, inv_ref, h_ref, wg_ref, wih_ref, whh_ref, bih_ref,
        bhh_ref, h0_ref, c0_ref, lwih_ref, lwhh_ref, lbih_ref, lbhh_ref,
        linw_ref, linb_ref, out_o):
    inv = inv_ref[:n]
    h3 = _gru_block(part_ref[0][:n], part_ref[1][:n], inv, h_ref[...],
                    wg_ref[...], wih_ref, whh_ref, bih_ref, bhh_ref)
    h0v = h0_ref[...]
    g = [h3 @ lwih_ref[k] + lbih_ref[k] + h0v @ lwhh_ref[k] + lbhh_ref[k]
         for k in range(4)]
    i_t = jax.nn.sigmoid(g[0])
    f_t = jax.nn.sigmoid(g[1])
    g_t = jnp.tanh(g[2])
    o_t = jax.nn.sigmoid(g[3])
    c_t = f_t * c0_ref[...] + i_t * g_t
    h_t = o_t * jnp.tanh(c_t)
    z = jnp.maximum(h_t, 0.0)
    out_o[...] = z @ linw_ref[...] + linb_ref[...]

  return pl.pallas_call(
      f, out_shape=jax.ShapeDtypeStruct((n, out_dim), jnp.float32),
  )(part, inv_full, h, w_g, wih, whh, bih, bhh, h0, c0, lwih, lwhh, lbih,
    lbhh, linw, linb)


def _adj(n, out_dim, out, out_t):
  bi, bj = 1024, 2048
  gi, gj = pl.cdiv(n, bi), pl.cdiv(n, bj)

  def f(a_ref, b_ref, adj_ref):
    adj_ref[...] = jax.nn.sigmoid(
        jnp.dot(a_ref[...], b_ref[...], preferred_element_type=jnp.float32))

  return pl.pallas_call(
      f,
      grid=(gi, gj),
      in_specs=[
          pl.BlockSpec((bi, out_dim), lambda i, j: (i, 0)),
          pl.BlockSpec((out_dim, bj), lambda i, j: (0, j)),
      ],
      out_specs=pl.BlockSpec((bi, bj), lambda i, j: (i, j)),
      out_shape=jax.ShapeDtypeStruct((n, n), jnp.float32),
  )(out, out_t)


def kernel(x, edge_index, edge_weight, h_0, c_0, ggc_w, gru_w_ih, gru_w_hh,
           gru_b_ih, gru_b_hh, lstm_w_ih, lstm_w_hh, lstm_b_ih, lstm_b_hh,
           lin_w, lin_b):
  n, hid = x.shape
  e = edge_weight.shape[0]
  out_dim = lin_w.shape[0]

  mc_real = pl.cdiv(e, MCW)
  mc_per_tile = pl.cdiv(mc_real, 32)
  mc_per_tile += mc_per_tile % 2
  mc_total = 32 * mc_per_tile
  e_pad = mc_total * MCW
  cnt_stride = (pl.cdiv(n, 16) + 127) // 128 * 128
  n_pad = 16 * cnt_stride

  src = edge_index[0]
  dst = edge_index[1]
  pad = e_pad - e
  src2 = jnp.pad(src, (0, pad)).reshape(mc_total, MCW)
  dst2 = jnp.pad(dst, (0, pad)).reshape(mc_total, MCW)
  ew2 = jnp.pad(edge_weight, (0, pad)).reshape(mc_total, MCW)
  val2 = jnp.pad(jnp.ones((e,), jnp.float32), (0, pad)).reshape(mc_total, MCW)

  wih = gru_w_ih.reshape(3, hid, hid).transpose(0, 2, 1)
  whh = gru_w_hh.reshape(3, hid, hid).transpose(0, 2, 1)
  bih = gru_b_ih.reshape(3, 1, hid)
  bhh = gru_b_hh.reshape(3, 1, hid)
  lwih = lstm_w_ih.reshape(4, hid, hid).transpose(0, 2, 1)
  lwhh = lstm_w_hh.reshape(4, hid, hid).transpose(0, 2, 1)
  lbih = lstm_b_ih.reshape(4, 1, hid)
  lbhh = lstm_b_hh.reshape(4, 1, hid)
  linw = lin_w.T
  linb = lin_b.reshape(1, out_dim)

  agg1 = _sc_agg(n, n_pad, hid, mc_total, mc_real, True, cnt_stride)
  agg23 = _sc_agg(n, n_pad, hid, mc_total, mc_real, False, cnt_stride)

  part1, cntp = agg1(src2, dst2, ew2, val2, x)
  cntp = cntp.reshape(2, 16 * cnt_stride, 1)
  h1, inv_full = _tc_layer1(n, hid, part1, cntp, x, ggc_w[0], wih, whh, bih,
                            bhh)
  (part2,) = agg23(src2, dst2, ew2, h1)
  h2 = _tc_layer2(n, hid, part2, inv_full, h1, ggc_w[1], wih, whh, bih, bhh)
  (part3,) = agg23(src2, dst2, ew2, h2)
  out = _tc_layer3(n, hid, out_dim, part3, inv_full, h2, ggc_w[2], wih, whh,
                   bih, bhh, h_0, c_0, lwih, lwhh, lbih, lbhh, linw, linb)
  adj = _adj(n, out_dim, out, out.T)
  return out, adj


# speedup vs baseline: 12.3802x; 1.2453x over previous
import functools

import jax
import jax.numpy as jnp
from jax import lax
from jax.experimental import pallas as pl
from jax.experimental.pallas import tpu as pltpu
from jax.experimental.pallas import tpu_sc as plsc

MCW = 128


@functools.lru_cache(maxsize=None)
def _sc_agg(n, n_pad, hid, mc_stage, mc_real, mc_per_tile, with_cnt,
            with_val, cnt_stride):
  mesh = plsc.VectorSubcoreMesh(core_axis_name="c", subcore_axis_name="s")
  NSLOT = 4
  assert mc_per_tile % NSLOT == 0 and mc_stage >= mc_per_tile
  rows_per_tile = n_pad // 16
  nh16 = hid // 16

  out_type = [jax.ShapeDtypeStruct((2, n_pad, hid), jnp.float32)]
  if with_cnt:
    out_type.append(
        jax.ShapeDtypeStruct((2, 1, 16 * cnt_stride), jnp.float32))

  scratch = (
      pltpu.VMEM((mc_per_tile, MCW), jnp.int32),
      pltpu.VMEM((mc_per_tile, MCW), jnp.int32),
      pltpu.VMEM((mc_per_tile, MCW), jnp.float32),
      pltpu.VMEM((mc_per_tile, MCW), jnp.float32),
      pltpu.VMEM((NSLOT, MCW, hid), jnp.float32),
      pltpu.VMEM((NSLOT, MCW, hid), jnp.float32),
      pltpu.VMEM((rows_per_tile, hid), jnp.float32),
      pltpu.VMEM((cnt_stride,), jnp.float32),
      pltpu.VMEM((MCW,), jnp.float32),
      pltpu.VMEM_SHARED((n_pad, hid), jnp.float32),
      pltpu.VMEM_SHARED((16 * cnt_stride,), jnp.float32),
  ) + (pltpu.SemaphoreType.DMA,) * (2 * NSLOT)

  def body(*refs):
    if with_val:
      (src2_h, dst2_h, ew2_h, val2_h, h_h, part_o) = refs[:6]
      k = 6
    else:
      (src2_h, dst2_h, ew2_h, h_h, part_o) = refs[:5]
      val2_h = None
      k = 5
    if with_cnt:
      cntp_o = refs[k]
      k += 1
    else:
      cntp_o = None
    (src_v, dst_v, ew_v, val_v, rows_v, msg_v, zero_v, zline_v, ones_v,
     acc_sp, cnt_sp) = refs[k:k + 11]
    sems = refs[k + 11:]
    gsems = sems[:NSLOT]
    ssems = sems[NSLOT:]

    c = lax.axis_index("c")
    s = lax.axis_index("s")
    wid = c * 16 + s
    mc0 = wid * mc_per_tile
    mc0c = jnp.minimum(mc0, mc_stage - mc_per_tile)
    off = mc0 - mc0c
    r_real = jnp.clip(mc_real - mc0, 0, mc_per_tile)

    zv = jnp.zeros((16,), jnp.float32)
    ov = jnp.ones((16,), jnp.float32)

    def zb(i, carry):
      for j in range(nh16):
        zero_v[i, pl.ds(16 * j, 16)] = zv
      return carry

    lax.fori_loop(0, rows_per_tile, zb, 0)
    pltpu.sync_copy(zero_v,
                    acc_sp.at[pl.ds(s * rows_per_tile, rows_per_tile), :])
    if with_cnt:
      def zc(i, carry):
        zline_v[pl.ds(16 * i, 16)] = zv
        return carry

      lax.fori_loop(0, cnt_stride // 16, zc, 0)
      pltpu.sync_copy(zline_v, cnt_sp.at[pl.ds(s * cnt_stride, cnt_stride)])
      for j in range(MCW // 16):
        ones_v[pl.ds(16 * j, 16)] = ov

    pltpu.sync_copy(src2_h.at[pl.ds(mc0c, mc_per_tile), :], src_v)
    pltpu.sync_copy(dst2_h.at[pl.ds(mc0c, mc_per_tile), :], dst_v)
    pltpu.sync_copy(ew2_h.at[pl.ds(mc0c, mc_per_tile), :], ew_v)
    if with_val:
      pltpu.sync_copy(val2_h.at[pl.ds(mc0c, mc_per_tile), :], val_v)

    plsc.subcore_barrier()

    for k0 in range(2):
      @pl.when(k0 < r_real)
      def _(k0=k0):
        pltpu.async_copy(h_h.at[src_v.at[k0 + off]], rows_v.at[k0],
                         gsems[k0])

    def do_mc(m, slot):
      @pl.when(m < r_real)
      def _():
        pltpu.make_async_copy(h_h.at[src_v.at[m + off]], rows_v.at[slot],
                              gsems[slot]).wait()

      g2 = (slot + 2) % NSLOT

      @pl.when(m + 2 < r_real)
      def _():
        pltpu.async_copy(h_h.at[src_v.at[m + 2 + off]], rows_v.at[g2],
                         gsems[g2])

      @pl.when((m >= NSLOT) & (m < r_real))
      def _():
        pltpu.make_async_copy(msg_v.at[slot],
                              acc_sp.at[dst_v.at[m - NSLOT + off]],
                              ssems[slot]).wait()

      @pl.when(m < r_real)
      def _():
        for b in range(MCW // 16):
          ew_vec = ew_v[m + off, pl.ds(16 * b, 16)]
          for l in range(16):
            e = 16 * b + l
            w = ew_vec[l]
            for j in range(nh16):
              msg_v[slot, e, pl.ds(16 * j, 16)] = (
                  rows_v[slot, e, pl.ds(16 * j, 16)] * w)
        pltpu.async_copy(msg_v.at[slot], acc_sp.at[dst_v.at[m + off]],
                         ssems[slot], add=True)
        if with_cnt:
          if with_val:
            pltpu.sync_copy(val_v.at[m + off], cnt_sp.at[dst_v.at[m + off]],
                            add=True)
          else:
            pltpu.sync_copy(ones_v, cnt_sp.at[dst_v.at[m + off]], add=True)

    def loop_body(i, carry):
      for k2 in range(NSLOT):
        do_mc(NSLOT * i + k2, k2)
      return carry

    lax.fori_loop(0, mc_per_tile // NSLOT, loop_body, 0)

    for slot in range(NSLOT):
      lastm = r_real - 1 - ((r_real - 1 - slot) % NSLOT)

      @pl.when(lastm >= 0)
      def _():
        pltpu.make_async_copy(
            msg_v.at[slot],
            acc_sp.at[dst_v.at[jnp.maximum(lastm, 0) + off]],
            ssems[slot]).wait()

    plsc.subcore_barrier()

    pltpu.sync_copy(acc_sp.at[pl.ds(s * rows_per_tile, rows_per_tile), :],
                    part_o.at[c, pl.ds(s * rows_per_tile, rows_per_tile), :])
    if with_cnt:
      @pl.when(s == 0)
      def _():
        pltpu.sync_copy(cnt_sp, cntp_o.at[c, 0])

  return pl.kernel(body, out_type=tuple(out_type), mesh=mesh,
                   scratch_types=scratch,
                   compiler_params=pltpu.CompilerParams(
                       use_tc_tiling_on_sc=False))


def _gru_block(part0, part1, inv, h, w_g, wih, whh, bih, bhh):
  agg = ((part0 + part1) * inv) @ w_g
  r = jax.nn.sigmoid(agg @ wih[0] + bih[0] + h @ whh[0] + bhh[0])
  z = jax.nn.sigmoid(agg @ wih[1] + bih[1] + h @ whh[1] + bhh[1])
  nn = jnp.tanh(agg @ wih[2] + bih[2] + r * (h @ whh[2] + bhh[2]))
  return (1.0 - z) * nn + z * h


def _tc_layer1(n, hid, part, cntp, h, w_g, wih, whh, bih, bhh):
  def f(part_ref, cntp_ref, h_ref, wg_ref, wih_ref, whh_ref, bih_ref,
        bhh_ref, h_out, inv_out):
    cnt = cntp_ref[0] + cntp_ref[1]
    inv_full = 1.0 / jnp.clip(cnt, 1.0, None)
    inv_out[...] = inv_full
    inv = inv_full[:n]
    h_out[...] = _gru_block(part_ref[0][:n], part_ref[1][:n], inv, h_ref[...],
                            wg_ref[...], wih_ref, whh_ref, bih_ref, bhh_ref)

  np_ = cntp.shape[1]
  return pl.pallas_call(
      f,
      out_shape=(jax.ShapeDtypeStruct((n, hid), jnp.float32),
                 jax.ShapeDtypeStruct((np_, 1), jnp.float32)),
  )(part, cntp, h, w_g, wih, whh, bih, bhh)


def _tc_layer2(n, hid, part, inv_full, h, w_g, wih, whh, bih, bhh):
  def f(part_ref, inv_ref, h_ref, wg_ref, wih_ref, whh_ref, bih_ref,
        bhh_ref, h_out):
    inv = inv_ref[:n]
    h_out[...] = _gru_block(part_ref[0][:n], part_ref[1][:n], inv, h_ref[...],
                            wg_ref[...], wih_ref, whh_ref, bih_ref, bhh_ref)

  return pl.pallas_call(
      f, out_shape=jax.ShapeDtypeStruct((n, hid), jnp.float32),
  )(part, inv_full, h, w_g, wih, whh, bih, bhh)


def _tc_layer3(n, hid, out_dim, part, inv_full, h, w_g, wih, whh, bih, bhh,
               h0, c0, lwih, lwhh, lbih, lbhh, linw, linb):
  def f(part_ref, inv_ref, h_ref, wg_ref, wih_ref, whh_ref, bih_ref,
        bhh_ref, h0_ref, c0_ref, lwih_ref, lwhh_ref, lbih_ref, lbhh_ref,
        linw_ref, linb_ref, out_o):
    inv = inv_ref[:n]
    h3 = _gru_block(part_ref[0][:n], part_ref[1][:n], inv, h_ref[...],
                    wg_ref[...], wih_ref, whh_ref, bih_ref, bhh_ref)
    h0v = h0_ref[...]
    g = [h3 @ lwih_ref[k] + lbih_ref[k] + h0v @ lwhh_ref[k] + lbhh_ref[k]
         for k in range(4)]
    i_t = jax.nn.sigmoid(g[0])
    f_t = jax.nn.sigmoid(g[1])
    g_t = jnp.tanh(g[2])
    o_t = jax.nn.sigmoid(g[3])
    c_t = f_t * c0_ref[...] + i_t * g_t
    h_t = o_t * jnp.tanh(c_t)
    z = jnp.maximum(h_t, 0.0)
    out_o[...] = z @ linw_ref[...] + linb_ref[...]

  return pl.pallas_call(
      f, out_shape=jax.ShapeDtypeStruct((n, out_dim), jnp.float32),
  )(part, inv_full, h, w_g, wih, whh, bih, bhh, h0, c0, lwih, lwhh, lbih,
    lbhh, linw, linb)


def _adj(n, out_dim, out, out_t):
  bi, bj = 1024, 2048
  gi, gj = pl.cdiv(n, bi), pl.cdiv(n, bj)

  def f(a_ref, b_ref, adj_ref):
    p = jnp.dot(a_ref[...], b_ref[...], preferred_element_type=jnp.float32)
    adj_ref[...] = 0.5 * jnp.tanh(0.5 * p) + 0.5

  return pl.pallas_call(
      f,
      grid=(gi, gj),
      in_specs=[
          pl.BlockSpec((bi, out_dim), lambda i, j: (i, 0)),
          pl.BlockSpec((out_dim, bj), lambda i, j: (0, j)),
      ],
      out_specs=pl.BlockSpec((bi, bj), lambda i, j: (i, j)),
      out_shape=jax.ShapeDtypeStruct((n, n), jnp.float32),
  )(out, out_t)


def kernel(x, edge_index, edge_weight, h_0, c_0, ggc_w, gru_w_ih, gru_w_hh,
           gru_b_ih, gru_b_hh, lstm_w_ih, lstm_w_hh, lstm_b_ih, lstm_b_hh,
           lin_w, lin_b):
  n, hid = x.shape
  e = edge_weight.shape[0]
  out_dim = lin_w.shape[0]

  mc_real = pl.cdiv(e, MCW)
  mc_per_tile = (pl.cdiv(mc_real, 32) + 3) // 4 * 4
  mc_stage = max(mc_real, mc_per_tile)
  cnt_stride = (pl.cdiv(n, 16) + 127) // 128 * 128
  n_pad = 16 * cnt_stride
  with_val = (e % MCW) != 0

  src = edge_index[0]
  dst = edge_index[1]
  pad = mc_stage * MCW - e
  if pad:
    src = jnp.pad(src, (0, pad))
    dst = jnp.pad(dst, (0, pad))
    edge_weight = jnp.pad(edge_weight, (0, pad))
  src2 = src.reshape(mc_stage, MCW)
  dst2 = dst.reshape(mc_stage, MCW)
  ew2 = edge_weight.reshape(mc_stage, MCW)
  val2 = None
  if with_val:
    val2 = jnp.pad(jnp.ones((e,), jnp.float32),
                   (0, pad)).reshape(mc_stage, MCW)

  wih = gru_w_ih.reshape(3, hid, hid).transpose(0, 2, 1)
  whh = gru_w_hh.reshape(3, hid, hid).transpose(0, 2, 1)
  bih = gru_b_ih.reshape(3, 1, hid)
  bhh = gru_b_hh.reshape(3, 1, hid)
  lwih = lstm_w_ih.reshape(4, hid, hid).transpose(0, 2, 1)
  lwhh = lstm_w_hh.reshape(4, hid, hid).transpose(0, 2, 1)
  lbih = lstm_b_ih.reshape(4, 1, hid)
  lbhh = lstm_b_hh.reshape(4, 1, hid)
  linw = lin_w.T
  linb = lin_b.reshape(1, out_dim)

  agg1 = _sc_agg(n, n_pad, hid, mc_stage, mc_real, mc_per_tile, True,
                 with_val, cnt_stride)
  agg23 = _sc_agg(n, n_pad, hid, mc_stage, mc_real, mc_per_tile, False,
                  False, cnt_stride)

  if with_val:
    part1, cntp = agg1(src2, dst2, ew2, val2, x)
  else:
    part1, cntp = agg1(src2, dst2, ew2, x)
  cntp = cntp.reshape(2, 16 * cnt_stride, 1)
  h1, inv_full = _tc_layer1(n, hid, part1, cntp, x, ggc_w[0], wih, whh, bih,
                            bhh)
  (part2,) = agg23(src2, dst2, ew2, h1)
  h2 = _tc_layer2(n, hid, part2, inv_full, h1, ggc_w[1], wih, whh, bih, bhh)
  (part3,) = agg23(src2, dst2, ew2, h2)
  out = _tc_layer3(n, hid, out_dim, part3, inv_full, h2, ggc_w[2], wih, whh,
                   bih, bhh, h_0, c_0, lwih, lwhh, lbih, lbhh, linw, linb)
  adj = _adj(n, out_dim, out, out.T)
  return out, adj


# speedup vs baseline: 15.0835x; 1.2184x over previous
import functools

import jax
import jax.numpy as jnp
from jax import lax
from jax.experimental import pallas as pl
from jax.experimental.pallas import tpu as pltpu
from jax.experimental.pallas import tpu_sc as plsc

MCW = 128


@functools.lru_cache(maxsize=None)
def _sc_agg(n, n_pad, hid, mc_stage, mc_real, mc_per_tile, with_cnt,
            with_val, cnt_stride):
  mesh = plsc.VectorSubcoreMesh(core_axis_name="c", subcore_axis_name="s")
  NSLOT = 4
  assert mc_per_tile % NSLOT == 0 and mc_stage >= mc_per_tile
  rows_per_tile = n_pad // 16
  nh16 = hid // 16

  out_type = [jax.ShapeDtypeStruct((2, n_pad, hid), jnp.float32)]
  if with_cnt:
    out_type.append(
        jax.ShapeDtypeStruct((2, 1, 16 * cnt_stride), jnp.float32))

  scratch = (
      pltpu.VMEM((mc_per_tile, MCW), jnp.int32),
      pltpu.VMEM((mc_per_tile, MCW), jnp.int32),
      pltpu.VMEM((mc_per_tile, MCW), jnp.float32),
      pltpu.VMEM((mc_per_tile, MCW), jnp.float32),
      pltpu.VMEM((NSLOT, MCW, hid), jnp.float32),
      pltpu.VMEM((NSLOT, MCW, hid), jnp.float32),
      pltpu.VMEM((rows_per_tile, hid), jnp.float32),
      pltpu.VMEM((cnt_stride,), jnp.float32),
      pltpu.VMEM((MCW,), jnp.float32),
      pltpu.VMEM_SHARED((n_pad, hid), jnp.float32),
      pltpu.VMEM_SHARED((16 * cnt_stride,), jnp.float32),
  ) + (pltpu.SemaphoreType.DMA,) * (2 * NSLOT)

  def body(*refs):
    if with_val:
      (src2_h, dst2_h, ew2_h, val2_h, h_h, part_o) = refs[:6]
      k = 6
    else:
      (src2_h, dst2_h, ew2_h, h_h, part_o) = refs[:5]
      val2_h = None
      k = 5
    if with_cnt:
      cntp_o = refs[k]
      k += 1
    else:
      cntp_o = None
    (src_v, dst_v, ew_v, val_v, rows_v, msg_v, zero_v, zline_v, ones_v,
     acc_sp, cnt_sp) = refs[k:k + 11]
    sems = refs[k + 11:]
    gsems = sems[:NSLOT]
    ssems = sems[NSLOT:]

    c = lax.axis_index("c")
    s = lax.axis_index("s")
    wid = c * 16 + s
    mc0 = wid * mc_per_tile
    mc0c = jnp.minimum(mc0, mc_stage - mc_per_tile)
    off = mc0 - mc0c
    r_real = jnp.clip(mc_real - mc0, 0, mc_per_tile)

    zv = jnp.zeros((16,), jnp.float32)
    ov = jnp.ones((16,), jnp.float32)

    def zb(i, carry):
      for j in range(nh16):
        zero_v[i, pl.ds(16 * j, 16)] = zv
      return carry

    lax.fori_loop(0, rows_per_tile, zb, 0)
    pltpu.sync_copy(zero_v,
                    acc_sp.at[pl.ds(s * rows_per_tile, rows_per_tile), :])
    if with_cnt:
      def zc(i, carry):
        zline_v[pl.ds(16 * i, 16)] = zv
        return carry

      lax.fori_loop(0, cnt_stride // 16, zc, 0)
      pltpu.sync_copy(zline_v, cnt_sp.at[pl.ds(s * cnt_stride, cnt_stride)])
      for j in range(MCW // 16):
        ones_v[pl.ds(16 * j, 16)] = ov

    pltpu.sync_copy(src2_h.at[pl.ds(mc0c, mc_per_tile), :], src_v)
    pltpu.sync_copy(dst2_h.at[pl.ds(mc0c, mc_per_tile), :], dst_v)
    pltpu.sync_copy(ew2_h.at[pl.ds(mc0c, mc_per_tile), :], ew_v)
    if with_val:
      pltpu.sync_copy(val2_h.at[pl.ds(mc0c, mc_per_tile), :], val_v)

    plsc.subcore_barrier()

    for k0 in range(2):
      @pl.when(k0 < r_real)
      def _(k0=k0):
        pltpu.async_copy(h_h.at[src_v.at[k0 + off]], rows_v.at[k0],
                         gsems[k0])

    def do_mc(m, slot):
      @pl.when(m < r_real)
      def _():
        pltpu.make_async_copy(h_h.at[src_v.at[m + off]], rows_v.at[slot],
                              gsems[slot]).wait()

      g2 = (slot + 2) % NSLOT

      @pl.when(m + 2 < r_real)
      def _():
        pltpu.async_copy(h_h.at[src_v.at[m + 2 + off]], rows_v.at[g2],
                         gsems[g2])

      @pl.when((m >= NSLOT) & (m < r_real))
      def _():
        pltpu.make_async_copy(msg_v.at[slot],
                              acc_sp.at[dst_v.at[m - NSLOT + off]],
                              ssems[slot]).wait()

      @pl.when(m < r_real)
      def _():
        for b in range(MCW // 16):
          ew_vec = ew_v[m + off, pl.ds(16 * b, 16)]
          for l in range(16):
            e = 16 * b + l
            w = ew_vec[l]
            for j in range(nh16):
              msg_v[slot, e, pl.ds(16 * j, 16)] = (
                  rows_v[slot, e, pl.ds(16 * j, 16)] * w)
        pltpu.async_copy(msg_v.at[slot], acc_sp.at[dst_v.at[m + off]],
                         ssems[slot], add=True)
        if with_cnt:
          if with_val:
            pltpu.sync_copy(val_v.at[m + off], cnt_sp.at[dst_v.at[m + off]],
                            add=True)
          else:
            pltpu.sync_copy(ones_v, cnt_sp.at[dst_v.at[m + off]], add=True)

    def loop_body(i, carry):
      for k2 in range(NSLOT):
        do_mc(NSLOT * i + k2, k2)
      return carry

    lax.fori_loop(0, mc_per_tile // NSLOT, loop_body, 0)

    for slot in range(NSLOT):
      lastm = r_real - 1 - ((r_real - 1 - slot) % NSLOT)

      @pl.when(lastm >= 0)
      def _():
        pltpu.make_async_copy(
            msg_v.at[slot],
            acc_sp.at[dst_v.at[jnp.maximum(lastm, 0) + off]],
            ssems[slot]).wait()

    plsc.subcore_barrier()

    pltpu.sync_copy(acc_sp.at[pl.ds(s * rows_per_tile, rows_per_tile), :],
                    part_o.at[c, pl.ds(s * rows_per_tile, rows_per_tile), :])
    if with_cnt:
      @pl.when(s == 0)
      def _():
        pltpu.sync_copy(cnt_sp, cntp_o.at[c, 0])

  return pl.kernel(body, out_type=tuple(out_type), mesh=mesh,
                   scratch_types=scratch,
                   compiler_params=pltpu.CompilerParams(
                       use_tc_tiling_on_sc=False))


PK = 4


def _gru_block4(p0, p1, inv4, h4, w_g4, wih4, whh4, bih4, bhh4):
  agg = ((p0 + p1) * inv4) @ w_g4
  r = jax.nn.sigmoid(agg @ wih4[0] + bih4[0] + h4 @ whh4[0] + bhh4[0])
  z = jax.nn.sigmoid(agg @ wih4[1] + bih4[1] + h4 @ whh4[1] + bhh4[1])
  nn = jnp.tanh(agg @ wih4[2] + bih4[2] + r * (h4 @ whh4[2] + bhh4[2]))
  return (1.0 - z) * nn + z * h4


def _tc_layer1(r, part4, cntp4, h4, w_g4, wih4, whh4, bih4, bhh4, e4):
  def f(part_ref, cntp_ref, h_ref, wg_ref, wih_ref, whh_ref, bih_ref,
        bhh_ref, e4_ref, h_out, inv_out):
    cnt4 = cntp_ref[0] + cntp_ref[1]
    inv4c = 1.0 / jnp.clip(cnt4, 1.0, None)
    inv_out[...] = inv4c
    inv4 = jnp.dot(inv4c[:r], e4_ref[...],
                   preferred_element_type=jnp.float32)
    h_out[...] = _gru_block4(part_ref[0][:r], part_ref[1][:r], inv4,
                             h_ref[...], wg_ref[...], wih_ref, whh_ref,
                             bih_ref, bhh_ref)

  rp = cntp4.shape[1]
  return pl.pallas_call(
      f,
      out_shape=(jax.ShapeDtypeStruct((r, PK * 32), jnp.float32),
                 jax.ShapeDtypeStruct((rp, PK), jnp.float32)),
  )(part4, cntp4, h4, w_g4, wih4, whh4, bih4, bhh4, e4)


def _tc_layer2(r, part4, inv4c, h4, w_g4, wih4, whh4, bih4, bhh4, e4):
  def f(part_ref, inv_ref, h_ref, wg_ref, wih_ref, whh_ref, bih_ref,
        bhh_ref, e4_ref, h_out):
    inv4 = jnp.dot(inv_ref[:r], e4_ref[...],
                   preferred_element_type=jnp.float32)
    h_out[...] = _gru_block4(part_ref[0][:r], part_ref[1][:r], inv4,
                             h_ref[...], wg_ref[...], wih_ref, whh_ref,
                             bih_ref, bhh_ref)

  return pl.pallas_call(
      f, out_shape=jax.ShapeDtypeStruct((r, PK * 32), jnp.float32),
  )(part4, inv4c, h4, w_g4, wih4, whh4, bih4, bhh4, e4)


def _tc_layer3(r, out_dim, part4, inv4c, h4, w_g4, wih4, whh4, bih4, bhh4,
               h04, c04, lwih4, lwhh4, lbih4, lbhh4, linw4, linb4, e4):
  def f(part_ref, inv_ref, h_ref, wg_ref, wih_ref, whh_ref, bih_ref,
        bhh_ref, h0_ref, c0_ref, lwih_ref, lwhh_ref, lbih_ref, lbhh_ref,
        linw_ref, linb_ref, e4_ref, out_o):
    inv4 = jnp.dot(inv_ref[:r], e4_ref[...],
                   preferred_element_type=jnp.float32)
    h3 = _gru_block4(part_ref[0][:r], part_ref[1][:r], inv4, h_ref[...],
                     wg_ref[...], wih_ref, whh_ref, bih_ref, bhh_ref)
    h0v = h0_ref[...]
    g = [h3 @ lwih_ref[k] + lbih_ref[k] + h0v @ lwhh_ref[k] + lbhh_ref[k]
         for k in range(4)]
    i_t = jax.nn.sigmoid(g[0])
    f_t = jax.nn.sigmoid(g[1])
    g_t = jnp.tanh(g[2])
    o_t = jax.nn.sigmoid(g[3])
    c_t = f_t * c0_ref[...] + i_t * g_t
    h_t = o_t * jnp.tanh(c_t)
    z = jnp.maximum(h_t, 0.0)
    out_o[...] = z @ linw_ref[...] + linb_ref[...]

  return pl.pallas_call(
      f, out_shape=jax.ShapeDtypeStruct((r, PK * out_dim), jnp.float32),
  )(part4, inv4c, h4, w_g4, wih4, whh4, bih4, bhh4, h04, c04, lwih4, lwhh4,
    lbih4, lbhh4, linw4, linb4, e4)


def _adj(n, out_dim, out, out_t):
  bi, bj = 1024, 2048
  gi, gj = pl.cdiv(n, bi), pl.cdiv(n, bj)

  def f(a_ref, b_ref, adj_ref):
    p = jnp.dot(a_ref[...], b_ref[...], preferred_element_type=jnp.float32)
    adj_ref[...] = 0.5 * jnp.tanh(0.5 * p) + 0.5

  return pl.pallas_call(
      f,
      grid=(gi, gj),
      in_specs=[
          pl.BlockSpec((bi, out_dim), lambda i, j: (i, 0)),
          pl.BlockSpec((out_dim, bj), lambda i, j: (0, j)),
      ],
      out_specs=pl.BlockSpec((bi, bj), lambda i, j: (i, j)),
      out_shape=jax.ShapeDtypeStruct((n, n), jnp.float32),
  )(out, out_t)


def kernel(x, edge_index, edge_weight, h_0, c_0, ggc_w, gru_w_ih, gru_w_hh,
           gru_b_ih, gru_b_hh, lstm_w_ih, lstm_w_hh, lstm_b_ih, lstm_b_hh,
           lin_w, lin_b):
  n, hid = x.shape
  e = edge_weight.shape[0]
  out_dim = lin_w.shape[0]
  assert n % PK == 0 and hid == 32

  mc_real = pl.cdiv(e, MCW)
  mc_per_tile = (pl.cdiv(mc_real, 32) + 3) // 4 * 4
  mc_stage = max(mc_real, mc_per_tile)
  cnt_stride = (pl.cdiv(n, 16) + 127) // 128 * 128
  n_pad = 16 * cnt_stride
  with_val = (e % MCW) != 0

  src = edge_index[0]
  dst = edge_index[1]
  pad = mc_stage * MCW - e
  if pad:
    src = jnp.pad(src, (0, pad))
    dst = jnp.pad(dst, (0, pad))
    edge_weight = jnp.pad(edge_weight, (0, pad))
  src2 = src.reshape(mc_stage, MCW)
  dst2 = dst.reshape(mc_stage, MCW)
  ew2 = edge_weight.reshape(mc_stage, MCW)
  val2 = None
  if with_val:
    val2 = jnp.pad(jnp.ones((e,), jnp.float32),
                   (0, pad)).reshape(mc_stage, MCW)

  eye4 = jnp.eye(PK, dtype=jnp.float32)
  wih = gru_w_ih.reshape(3, hid, hid).transpose(0, 2, 1)
  whh = gru_w_hh.reshape(3, hid, hid).transpose(0, 2, 1)
  wih4 = jnp.stack([jnp.kron(eye4, wih[g]) for g in range(3)])
  whh4 = jnp.stack([jnp.kron(eye4, whh[g]) for g in range(3)])
  bih = gru_b_ih.reshape(3, hid)
  bhh = gru_b_hh.reshape(3, hid)
  bih4 = jnp.stack([jnp.tile(bih[g], PK)[None, :] for g in range(3)])
  bhh4 = jnp.stack([jnp.tile(bhh[g], PK)[None, :] for g in range(3)])
  wg4 = jnp.stack([jnp.kron(eye4, ggc_w[i]) for i in range(3)])
  lwih = lstm_w_ih.reshape(4, hid, hid).transpose(0, 2, 1)
  lwhh = lstm_w_hh.reshape(4, hid, hid).transpose(0, 2, 1)
  lwih4 = jnp.stack([jnp.kron(eye4, lwih[g]) for g in range(4)])
  lwhh4 = jnp.stack([jnp.kron(eye4, lwhh[g]) for g in range(4)])
  lbih = lstm_b_ih.reshape(4, hid)
  lbhh = lstm_b_hh.reshape(4, hid)
  lbih4 = jnp.stack([jnp.tile(lbih[g], PK)[None, :] for g in range(4)])
  lbhh4 = jnp.stack([jnp.tile(lbhh[g], PK)[None, :] for g in range(4)])
  linw4 = jnp.kron(eye4, lin_w.T)
  linb4 = jnp.tile(lin_b, PK).reshape(1, PK * out_dim)
  lanes = jnp.arange(PK * hid) // hid
  e4 = (lanes[None, :] == jnp.arange(PK)[:, None]).astype(jnp.float32)

  r = n // PK
  rp = n_pad // PK
  x4 = x.reshape(r, PK * hid)
  h04 = h_0.reshape(r, PK * hid)
  c04 = c_0.reshape(r, PK * hid)

  agg1 = _sc_agg(n, n_pad, hid, mc_stage, mc_real, mc_per_tile, True,
                 with_val, cnt_stride)
  agg23 = _sc_agg(n, n_pad, hid, mc_stage, mc_real, mc_per_tile, False,
                  False, cnt_stride)

  if with_val:
    part1, cntp = agg1(src2, dst2, ew2, val2, x)
  else:
    part1, cntp = agg1(src2, dst2, ew2, x)
  part14 = part1.reshape(2, rp, PK * hid)
  cntp4 = cntp.reshape(2, rp, PK)
  h14, inv4c = _tc_layer1(r, part14, cntp4, x4, wg4[0], wih4, whh4, bih4,
                          bhh4, e4)
  h1 = h14.reshape(n, hid)
  (part2,) = agg23(src2, dst2, ew2, h1)
  h24 = _tc_layer2(r, part2.reshape(2, rp, PK * hid), inv4c, h14, wg4[1],
                   wih4, whh4, bih4, bhh4, e4)
  h2 = h24.reshape(n, hid)
  (part3,) = agg23(src2, dst2, ew2, h2)
  out4 = _tc_layer3(r, out_dim, part3.reshape(2, rp, PK * hid), inv4c, h24,
                    wg4[2], wih4, whh4, bih4, bhh4, h04, c04, lwih4, lwhh4,
                    lbih4, lbhh4, linw4, linb4, e4)
  out = out4.reshape(n, out_dim)
  adj = _adj(n, out_dim, out, out.T)
  return out, adj
